# baseline (speedup 1.0000x reference)
import functools

import jax
import jax.numpy as jnp
from jax import lax
from jax.experimental import pallas as pl
from jax.experimental.pallas import tpu as pltpu

_VMEM_LIMIT = 48 * 1024 * 1024


def _cp(*sem):
    return pltpu.CompilerParams(dimension_semantics=sem,
                                vmem_limit_bytes=_VMEM_LIMIT)


def _conv_kernel(*refs, B, C, P, W, fused_prologue, in_rows):
    if fused_prologue:
        x_ref, w_ref, m_ref, scale_ref, shift_ref, y_ref, stats_ref, xbuf, patch = refs
    else:
        x_ref, w_ref, m_ref, y_ref, stats_ref, xbuf, patch = refs
        scale_ref = shift_ref = None
    MARG = 35
    BP = B * P

    xbuf[:, 0:MARG] = jnp.zeros((C, MARG), xbuf.dtype)
    xbuf[:, MARG + BP:MARG + BP + MARG] = jnp.zeros((C, MARG), xbuf.dtype)

    if in_rows:
        for b in range(B):
            xc = x_ref[b * C:(b + 1) * C, :]
            xbuf[:, MARG + b * P:MARG + (b + 1) * P] = xc.astype(xbuf.dtype)
    else:
        xc = x_ref[0]
        if fused_prologue:
            xc = jnp.maximum(xc.astype(jnp.float32) * scale_ref[...]
                             + shift_ref[...], 0.0)
        xbuf[:, MARG:MARG + BP] = xc.astype(xbuf.dtype)

    for kh in range(3):
        for kw in range(3):
            t = kh * 3 + kw
            s = (kh - 1) * W + (kw - 1)
            patch[t * C:(t + 1) * C, :] = (
                xbuf[:, MARG + s:MARG + s + BP] * m_ref[t:t + 1, :])

    acc = jnp.dot(w_ref[...], patch[...], preferred_element_type=jnp.float32)
    y_ref[...] = acc.reshape(1, C, BP).astype(y_ref.dtype)
    stats_ref[0, :, 0:1] = jnp.sum(acc, axis=1, keepdims=True)
    stats_ref[0, :, 1:2] = jnp.sum(acc * acc, axis=1, keepdims=True)


def _conv3x3_bn_stats(x, w_t, masks, *, N, C, P, W, block_b, prologue=None):
    G = N // block_b
    BP = block_b * P
    first = x.ndim == 2
    in_specs = [
        pl.BlockSpec((block_b * C, P), lambda n: (n, 0)) if first
        else pl.BlockSpec((1, C, BP), lambda n: (n, 0, 0)),
        pl.BlockSpec((C, 9 * C), lambda n: (0, 0)),
        pl.BlockSpec((9, BP), lambda n: (0, 0)),
    ]
    args = [x, w_t, masks]
    if prologue is not None:
        scale, shift = prologue
        in_specs += [pl.BlockSpec((C, BP), lambda n: (0, 0)),
                     pl.BlockSpec((C, BP), lambda n: (0, 0))]
        args += [jnp.broadcast_to(scale.astype(jnp.float32)[:, None], (C, BP)),
                 jnp.broadcast_to(shift.astype(jnp.float32)[:, None], (C, BP))]

    kern = functools.partial(_conv_kernel, B=block_b, C=C, P=P, W=W,
                             fused_prologue=prologue is not None,
                             in_rows=first)
    y, stats = pl.pallas_call(
        kern,
        out_shape=(jax.ShapeDtypeStruct((G, C, BP), jnp.bfloat16),
                   jax.ShapeDtypeStruct((G, C, 2), jnp.float32)),
        grid=(G,),
        in_specs=in_specs,
        out_specs=(pl.BlockSpec((1, C, BP), lambda n: (n, 0, 0)),
                   pl.BlockSpec((1, C, 2), lambda n: (n, 0, 0))),
        scratch_shapes=[
            pltpu.VMEM((C, 35 + BP + 35), jnp.bfloat16),
            pltpu.VMEM((9 * C, BP), jnp.bfloat16),
        ],
        compiler_params=_cp("parallel"),
    )(*args)
    return y, stats


def _tap_masks(B, H, W):
    p = jnp.arange(B * H * W)
    h = (p // W) % H
    w = p % W
    rows = []
    for kh in range(3):
        for kw in range(3):
            ok = ((h + kh - 1 >= 0) & (h + kh - 1 < H)
                  & (w + kw - 1 >= 0) & (w + kw - 1 < W))
            rows.append(ok)
    return jnp.stack(rows).astype(jnp.bfloat16)


def _bn_scale_shift(stats, gamma, beta, count, eps=1e-5):
    s = jnp.sum(stats, axis=0)
    mean = s[:, 0] / count
    var = s[:, 1] / count - mean * mean
    scale = gamma * lax.rsqrt(var + eps)
    shift = beta - mean * scale
    return scale, shift


def _finalize_kernel(y_ref, skip_ref, scale_ref, shift_ref, o_ref, *, B, C, P):
    y = y_ref[0].astype(jnp.float32)
    o = y * scale_ref[...] + shift_ref[...]
    for b in range(B):
        ob = o[:, b * P:(b + 1) * P] + skip_ref[b * C:(b + 1) * C, :]
        o_ref[b * C:(b + 1) * C, :] = jnp.maximum(ob, 0.0)


def _finalize(y2, skipf, scale, shift, *, N, C, P, block_b):
    G = N // block_b
    BP = block_b * P
    out = pl.pallas_call(
        functools.partial(_finalize_kernel, B=block_b, C=C, P=P),
        out_shape=jax.ShapeDtypeStruct((N * C, P), jnp.float32),
        grid=(G,),
        in_specs=[pl.BlockSpec((1, C, BP), lambda i: (i, 0, 0)),
                  pl.BlockSpec((block_b * C, P), lambda i: (i, 0)),
                  pl.BlockSpec((C, BP), lambda i: (0, 0)),
                  pl.BlockSpec((C, BP), lambda i: (0, 0))],
        out_specs=pl.BlockSpec((block_b * C, P), lambda i: (i, 0)),
        compiler_params=_cp("parallel"),
    )(y2, skipf,
      jnp.broadcast_to(scale.astype(jnp.float32)[:, None], (C, BP)),
      jnp.broadcast_to(shift.astype(jnp.float32)[:, None], (C, BP)))
    return out


def kernel(x, w1, w2, g1, beta1, g2, beta2):
    N, C, H, W = x.shape
    P = H * W
    B = 4
    xf = x.reshape(N * C, P)
    w1t = jnp.transpose(w1.reshape(9 * C, C), (1, 0)).astype(jnp.bfloat16)
    w2t = jnp.transpose(w2.reshape(9 * C, C), (1, 0)).astype(jnp.bfloat16)
    masks = _tap_masks(B, H, W)

    y1, st1 = _conv3x3_bn_stats(xf, w1t, masks, N=N, C=C, P=P, W=W, block_b=B)
    scale1, shift1 = _bn_scale_shift(st1, g1, beta1, N * P)

    y2, st2 = _conv3x3_bn_stats(y1, w2t, masks, N=N, C=C, P=P, W=W, block_b=B,
                                prologue=(scale1, shift1))
    scale2, shift2 = _bn_scale_shift(st2, g2, beta2, N * P)

    out = _finalize(y2, xf, scale2, shift2, N=N, C=C, P=P, block_b=B)
    return out.reshape(N, C, H, W)


# speedup vs baseline: 2.6415x; 2.6415x over previous
import functools

import jax
import jax.numpy as jnp
from jax import lax
from jax.experimental import pallas as pl
from jax.experimental.pallas import tpu as pltpu

_VMEM_LIMIT = 48 * 1024 * 1024


def _cp(*sem):
    return pltpu.CompilerParams(dimension_semantics=sem,
                                vmem_limit_bytes=_VMEM_LIMIT)


def _conv_kernel(*refs, B, H, W, C, fused_prologue):
    if fused_prologue:
        x_ref, w_ref, scale_ref, shift_ref, y_ref, stats_ref, xpad = refs
    else:
        x_ref, w_ref, y_ref, stats_ref, xpad = refs
        scale_ref = shift_ref = None
    Hp, Wp = H + 2, W + 2
    P = H * W

    xpad[0:1, :, :] = jnp.zeros((1, Wp, C), xpad.dtype)
    xpad[Hp - 1:Hp, :, :] = jnp.zeros((1, Wp, C), xpad.dtype)
    xpad[:, 0:1, :] = jnp.zeros((Hp, 1, C), xpad.dtype)
    xpad[:, Wp - 1:Wp, :] = jnp.zeros((Hp, 1, C), xpad.dtype)

    ssum = jnp.zeros((1, C), jnp.float32)
    ssq = jnp.zeros((1, C), jnp.float32)
    for b in range(B):
        xin = x_ref[b].astype(jnp.float32)
        if fused_prologue:
            xin = jnp.maximum(xin * scale_ref[...] + shift_ref[...], 0.0)
        xpad[1:H + 1, 1:W + 1, :] = xin
        acc = jnp.zeros((P, C), jnp.float32)
        for kh in range(3):
            for kw in range(3):
                t = kh * 3 + kw
                acc = acc + jnp.dot(
                    xpad[kh:kh + H, kw:kw + W, :].reshape(P, C),
                    w_ref[t * C:(t + 1) * C, :],
                    preferred_element_type=jnp.float32)
        y_ref[b] = acc.reshape(H, W, C).astype(y_ref.dtype)
        ssum = ssum + jnp.sum(acc, axis=0, keepdims=True)
        ssq = ssq + jnp.sum(acc * acc, axis=0, keepdims=True)
    stats_ref[0, 0:1, :] = ssum
    stats_ref[0, 1:2, :] = ssq


def _conv3x3_bn_stats(x, w_flat, *, shape_nhwc, block_b, prologue=None):
    N, H, W, C = shape_nhwc
    G = N // block_b
    in_specs = [
        pl.BlockSpec((block_b, H, W, C), lambda n: (n, 0, 0, 0)),
        pl.BlockSpec((9 * C, C), lambda n: (0, 0)),
    ]
    args = [x, w_flat]
    if prologue is not None:
        scale, shift = prologue
        in_specs += [pl.BlockSpec((1, C), lambda n: (0, 0)),
                     pl.BlockSpec((1, C), lambda n: (0, 0))]
        args += [scale.astype(jnp.float32).reshape(1, C),
                 shift.astype(jnp.float32).reshape(1, C)]

    kern = functools.partial(_conv_kernel, B=block_b, H=H, W=W, C=C,
                             fused_prologue=prologue is not None)
    y, stats = pl.pallas_call(
        kern,
        out_shape=(jax.ShapeDtypeStruct((N, H, W, C), jnp.bfloat16),
                   jax.ShapeDtypeStruct((G, 2, C), jnp.float32)),
        grid=(G,),
        in_specs=in_specs,
        out_specs=(pl.BlockSpec((block_b, H, W, C), lambda n: (n, 0, 0, 0)),
                   pl.BlockSpec((1, 2, C), lambda n: (n, 0, 0))),
        scratch_shapes=[pltpu.VMEM((H + 2, W + 2, C), jnp.float32)],
        compiler_params=_cp("parallel"),
    )(*args)
    return y, stats


def _bn_scale_shift(stats, gamma, beta, count, eps=1e-5):
    s = jnp.sum(stats, axis=0)
    mean = s[0] / count
    var = s[1] / count - mean * mean
    scale = gamma * lax.rsqrt(var + eps)
    shift = beta - mean * scale
    return scale, shift


def _finalize_kernel(y_ref, skip_ref, scale_ref, shift_ref, o_ref):
    y = y_ref[...].astype(jnp.float32)
    o = y * scale_ref[...] + shift_ref[...] + skip_ref[...].astype(jnp.float32)
    o_ref[...] = jnp.maximum(o, 0.0).astype(o_ref.dtype)


def _finalize(y2, skip, scale, shift, rows_block=8192):
    N, H, W, C = y2.shape
    rows = N * H * W
    while rows % rows_block:
        rows_block //= 2
    out = pl.pallas_call(
        _finalize_kernel,
        out_shape=jax.ShapeDtypeStruct((rows, C), jnp.bfloat16),
        grid=(rows // rows_block,),
        in_specs=[pl.BlockSpec((rows_block, C), lambda i: (i, 0)),
                  pl.BlockSpec((rows_block, C), lambda i: (i, 0)),
                  pl.BlockSpec((1, C), lambda i: (0, 0)),
                  pl.BlockSpec((1, C), lambda i: (0, 0))],
        out_specs=pl.BlockSpec((rows_block, C), lambda i: (i, 0)),
        compiler_params=_cp("parallel"),
    )(y2.reshape(rows, C), skip.reshape(rows, C),
      scale.astype(jnp.float32).reshape(1, C),
      shift.astype(jnp.float32).reshape(1, C))
    return out.reshape(N, H, W, C)


def kernel(x, w1, w2, g1, beta1, g2, beta2):
    N, C, H, W = x.shape
    P = H * W
    xh = jnp.transpose(x, (0, 2, 3, 1))
    w1f = w1.reshape(9 * C, C)
    w2f = w2.reshape(9 * C, C)

    y1, st1 = _conv3x3_bn_stats(xh, w1f, shape_nhwc=(N, H, W, C), block_b=8)
    scale1, shift1 = _bn_scale_shift(st1, g1, beta1, N * P)

    y2, st2 = _conv3x3_bn_stats(y1, w2f, shape_nhwc=(N, H, W, C), block_b=8,
                                prologue=(scale1, shift1))
    scale2, shift2 = _bn_scale_shift(st2, g2, beta2, N * P)

    out = _finalize(y2, xh, scale2, shift2)
    return jnp.transpose(out, (0, 3, 1, 2)).astype(jnp.float32)
